# Initial kernel scaffold; baseline (speedup 1.0000x reference)
import jax
import jax.numpy as jnp
from jax.experimental import pallas as pl


def kernel(input, batch, weight, bias):
    raise NotImplementedError("write your pallas kernel here")


# speedup vs baseline: 2.2284x; 2.2284x over previous
import functools

import numpy as np
import jax
import jax.numpy as jnp
from jax.experimental import pallas as pl

N = 100000
D = 208
G = 512
NSCAL = 32
EPS = 1e-5
R = 512
NBLK = -(-N // R)

_HI = jax.lax.Precision.HIGHEST


def _build_col_maps():
    gid = np.zeros((D,), np.int32)
    dcol = np.zeros((D,), np.float32)
    c = 0
    f = 0
    for mul, l in ((32, 0), (32, 1), (16, 2)):
        d = 2 * l + 1
        for m in range(mul):
            for j in range(d):
                gid[c] = f
                dcol[c] = d
                c += 1
            f += 1
    M = (gid[:, None] == gid[None, :]).astype(np.float32) / dcol[None, :]
    return M


_M_CONST = _build_col_maps()


def _stats_kernel(x_ref, b_ref, o_ref):
    step = pl.program_id(0)

    @pl.when(step == 0)
    def _():
        o_ref[...] = jnp.zeros_like(o_ref)

    x = x_ref[...]
    b = b_ref[...]
    rows = step * R + jax.lax.broadcasted_iota(jnp.int32, (R, 1), 0)
    valid = rows < N
    x = jnp.where(valid, x, 0.0)
    gids = jax.lax.broadcasted_iota(jnp.int32, (R, G), 1)
    oh = jnp.where((b == gids) & valid, 1.0, 0.0)
    vf = jnp.where(valid, 1.0, 0.0)
    contrib = jnp.concatenate(
        [x[:, :NSCAL], x * x, vf, jnp.zeros((R, 15), jnp.float32)], axis=1)
    o_ref[...] += jax.lax.dot_general(
        oh, contrib, (((0,), (0,)), ((), ())),
        preferred_element_type=jnp.float32, precision=_HI)


def _coeff_kernel(s_ref, m_ref, w_ref, bb_ref, scale_ref, shift_ref):
    s = s_ref[...]
    cnt = jnp.maximum(s[:, 240:241], 1.0)
    mean = s[:, :NSCAL] / cnt
    sq = s[:, NSCAL:NSCAL + D]
    gs = jax.lax.dot_general(
        sq, m_ref[...], (((1,), (0,)), ((), ())),
        preferred_element_type=jnp.float32, precision=_HI) / cnt
    meanp = jnp.concatenate([mean, jnp.zeros((G, D - NSCAL), jnp.float32)], 1)
    var = gs - meanp * meanp
    scale = w_ref[...] * jax.lax.rsqrt(var + EPS)
    scale_ref[...] = scale
    shift_ref[...] = bb_ref[...] - mean * scale[:, :NSCAL]


def _apply_kernel(x_ref, b_ref, scale_ref, shift_ref, o_ref):
    x = x_ref[...]
    b = b_ref[...]
    gids = jax.lax.broadcasted_iota(jnp.int32, (R, G), 1)
    oh = jnp.where(b == gids, 1.0, 0.0)
    rs = jax.lax.dot_general(
        oh, scale_ref[...], (((1,), (0,)), ((), ())),
        preferred_element_type=jnp.float32, precision=_HI)
    rsh = jax.lax.dot_general(
        oh, shift_ref[...], (((1,), (0,)), ((), ())),
        preferred_element_type=jnp.float32, precision=_HI)
    o_ref[...] = x * rs + jnp.concatenate(
        [rsh, jnp.zeros((R, D - NSCAL), jnp.float32)], axis=1)


@jax.jit
def kernel(input, batch, weight, bias):
    b2 = batch.astype(jnp.int32).reshape(N, 1)
    wcol = jnp.concatenate(
        [weight[:32], jnp.repeat(weight[32:64], 3), jnp.repeat(weight[64:], 5)]
    ).reshape(1, D)
    bb = bias.reshape(1, NSCAL)
    m = jnp.asarray(_M_CONST)

    stats = pl.pallas_call(
        _stats_kernel,
        grid=(NBLK,),
        in_specs=[
            pl.BlockSpec((R, D), lambda i: (i, 0)),
            pl.BlockSpec((R, 1), lambda i: (i, 0)),
        ],
        out_specs=pl.BlockSpec((G, 256), lambda i: (0, 0)),
        out_shape=jax.ShapeDtypeStruct((G, 256), jnp.float32),
    )(input, b2)

    scale, shift = pl.pallas_call(
        _coeff_kernel,
        out_shape=(
            jax.ShapeDtypeStruct((G, D), jnp.float32),
            jax.ShapeDtypeStruct((G, NSCAL), jnp.float32),
        ),
    )(stats, m, wcol, bb)

    out = pl.pallas_call(
        _apply_kernel,
        grid=(NBLK,),
        in_specs=[
            pl.BlockSpec((R, D), lambda i: (i, 0)),
            pl.BlockSpec((R, 1), lambda i: (i, 0)),
            pl.BlockSpec((G, D), lambda i: (0, 0)),
            pl.BlockSpec((G, NSCAL), lambda i: (0, 0)),
        ],
        out_specs=pl.BlockSpec((R, D), lambda i: (i, 0)),
        out_shape=jax.ShapeDtypeStruct((N, D), jnp.float32),
    )(input, b2, scale, shift)
    return out
